# baseline (speedup 1.0000x reference)
import jax
import jax.numpy as jnp
from jax import lax
from jax.experimental import pallas as pl
from jax.experimental.pallas import tpu as pltpu
from jax.experimental.pallas import tpu_sc as plsc

N = 10000
D = 128
H = 64
E = 320000
MLP = 128

NC = 2
NS = 16
NW = NC * NS
EP = E // NW
CH = 128
NCH = 79
EPP = NCH * CH
NPAD = 10112
ZR = NPAD // NS
DEGW = 8

_mesh = plsc.VectorSubcoreMesh(core_axis_name="c", subcore_axis_name="s")
_sc_params = pltpu.CompilerParams(use_tc_tiling_on_sc=False)


def _dot3(a, b):
    ah = a.astype(jnp.bfloat16)
    al = (a - ah.astype(jnp.float32)).astype(jnp.bfloat16)
    bh = b.astype(jnp.bfloat16)
    bl = (b - bh.astype(jnp.float32)).astype(jnp.bfloat16)

    def d(u, v):
        return jnp.dot(u, v, preferred_element_type=jnp.float32)

    return d(ah, bh) + d(ah, bl) + d(al, bh)


def _dotd(a, b):
    return jnp.dot(a, b, preferred_element_type=jnp.float32)


def _sc_deg_body(sd_hbm, dd_hbm, ones_hbm, z8_hbm, out_hbm,
                 idx_s, idx_d, ones_v, acc_out, acc_in):
    c = lax.axis_index("c")
    s = lax.axis_index("s")
    w = c * NS + s
    pltpu.sync_copy(sd_hbm.at[w], idx_s)
    pltpu.sync_copy(dd_hbm.at[w], idx_d)
    pltpu.sync_copy(ones_hbm, ones_v)
    pltpu.sync_copy(z8_hbm.at[pl.ds(s * ZR, ZR)], acc_out.at[pl.ds(s * ZR, ZR)])
    pltpu.sync_copy(z8_hbm.at[pl.ds(s * ZR, ZR)], acc_in.at[pl.ds(s * ZR, ZR)])
    plsc.subcore_barrier()

    def body(j, carry):
        pltpu.sync_copy(ones_v, acc_out.at[idx_s.at[j]], add=True)
        pltpu.sync_copy(ones_v, acc_in.at[idx_d.at[j]], add=True)
        return carry

    lax.fori_loop(0, NCH, body, 0)
    plsc.subcore_barrier()
    pltpu.sync_copy(acc_out.at[pl.ds(s * ZR, ZR)],
                    out_hbm.at[c, 0, pl.ds(s * ZR, ZR)])
    pltpu.sync_copy(acc_in.at[pl.ds(s * ZR, ZR)],
                    out_hbm.at[c, 1, pl.ds(s * ZR, ZR)])


_deg_call = pl.kernel(
    _sc_deg_body,
    out_type=jax.ShapeDtypeStruct((NC, 2, NPAD, DEGW), jnp.float32),
    mesh=_mesh,
    scratch_types=[
        pltpu.VMEM((NCH, CH), jnp.int32),
        pltpu.VMEM((NCH, CH), jnp.int32),
        pltpu.VMEM((CH, DEGW), jnp.float32),
        pltpu.VMEM_SHARED((NPAD, DEGW), jnp.float32),
        pltpu.VMEM_SHARED((NPAD, DEGW), jnp.float32),
    ],
    compiler_params=_sc_params,
)


def _sc_agg_body(h_hbm, sg_hbm, dd_hbm, z64_hbm, out_hbm,
                 idx_s, idx_d, gbuf, acc, sem):
    c = lax.axis_index("c")
    s = lax.axis_index("s")
    w = c * NS + s
    pltpu.sync_copy(sg_hbm.at[w], idx_s)
    pltpu.sync_copy(dd_hbm.at[w], idx_d)
    pltpu.sync_copy(z64_hbm.at[pl.ds(s * ZR, ZR)], acc.at[pl.ds(s * ZR, ZR)])
    plsc.subcore_barrier()

    def body(j, carry):
        pltpu.async_copy(h_hbm.at[idx_s.at[j]], gbuf, sem).wait()
        pltpu.sync_copy(gbuf, acc.at[idx_d.at[j]], add=True)
        return carry

    lax.fori_loop(0, NCH, body, 0)
    plsc.subcore_barrier()
    pltpu.sync_copy(acc.at[pl.ds(s * ZR, ZR)], out_hbm.at[c, pl.ds(s * ZR, ZR)])


_agg_call = pl.kernel(
    _sc_agg_body,
    out_type=jax.ShapeDtypeStruct((NC, NPAD, H), jnp.float32),
    mesh=_mesh,
    scratch_types=[
        pltpu.VMEM((NCH, CH), jnp.int32),
        pltpu.VMEM((NCH, CH), jnp.int32),
        pltpu.VMEM((CH, H), jnp.float32),
        pltpu.VMEM_SHARED((NPAD, H), jnp.float32),
        pltpu.SemaphoreType.DMA,
    ],
    compiler_params=_sc_params,
)

R = 1000
G = N // R


def _tc_norm_mm1_body(degp_ref, x_ref, w1_ref, hn_ref, ns_ref, nd_ref):
    dp = degp_ref[...]
    deg_out = dp[0, 0, :, :1] + dp[1, 0, :, :1]
    deg_in = dp[0, 1, :, :1] + dp[1, 1, :, :1]
    ns = lax.rsqrt(jnp.maximum(deg_out, 1.0))
    nd = lax.rsqrt(jnp.maximum(deg_in, 1.0))
    h = _dot3(x_ref[...], w1_ref[...])
    hn_ref[...] = h * ns
    ns_ref[...] = ns
    nd_ref[...] = nd


def _tc_norm_mm1(degp, x, W1):
    return pl.pallas_call(
        _tc_norm_mm1_body,
        grid=(G,),
        in_specs=[
            pl.BlockSpec((NC, 2, R, DEGW), lambda i: (0, 0, i, 0)),
            pl.BlockSpec((R, D), lambda i: (i, 0)),
            pl.BlockSpec((D, H), lambda i: (0, 0)),
        ],
        out_specs=[
            pl.BlockSpec((R, H), lambda i: (i, 0)),
            pl.BlockSpec((R, 1), lambda i: (i, 0)),
            pl.BlockSpec((R, 1), lambda i: (i, 0)),
        ],
        out_shape=[
            jax.ShapeDtypeStruct((N, H), jnp.float32),
            jax.ShapeDtypeStruct((N, 1), jnp.float32),
            jax.ShapeDtypeStruct((N, 1), jnp.float32),
        ],
    )(degp, x, W1)


def _tc_layer_body(agg_ref, nd_ref, ns_ref, b_ref, w_ref, out_ref):
    a = agg_ref[0] + agg_ref[1]
    h = jnp.maximum(a * nd_ref[...] + b_ref[...][None, :], 0.0)
    out_ref[...] = _dot3(h, w_ref[...]) * ns_ref[...]


def _tc_layer(agg, nd, ns, b, W):
    return pl.pallas_call(
        _tc_layer_body,
        grid=(G,),
        in_specs=[
            pl.BlockSpec((NC, R, H), lambda i: (0, i, 0)),
            pl.BlockSpec((R, 1), lambda i: (i, 0)),
            pl.BlockSpec((R, 1), lambda i: (i, 0)),
            pl.BlockSpec((H,), lambda i: (0,)),
            pl.BlockSpec((H, H), lambda i: (0, 0)),
        ],
        out_specs=pl.BlockSpec((R, H), lambda i: (i, 0)),
        out_shape=jax.ShapeDtypeStruct((N, H), jnp.float32),
    )(agg, nd, ns, b, W)


def _tc_head_body(agg_ref, nd_ref, b3_ref, m1_ref, c1_ref, m2_ref, c2_ref,
                  m3_ref, c3_ref, out_ref):
    a = agg_ref[0, :N] + agg_ref[1, :N]
    h = jnp.maximum(a * nd_ref[...] + b3_ref[...][None, :], 0.0)
    emb = jnp.sum(h, axis=0, keepdims=True) * (1.0 / N)
    z = jnp.maximum(
        _dot3(emb, m1_ref[...])
        + c1_ref[...][None, :], 0.0)
    z = jnp.maximum(
        _dot3(z, m2_ref[...])
        + c2_ref[...][None, :], 0.0)
    out_ref[...] = (_dot3(z, m3_ref[...])
                    + c3_ref[...][None, :])


def _tc_head(agg, nd, b3, M1, c1, M2, c2, M3, c3):
    return pl.pallas_call(
        _tc_head_body,
        out_shape=jax.ShapeDtypeStruct((1, 1), jnp.float32),
    )(agg, nd, b3, M1, c1, M2, c2, M3, c3)


def kernel(x, W1, b1, W2, b2, W3, b3, M1, c1, M2, c2, M3, c3, edge_index):
    src = edge_index[0].reshape(NW, EP)
    dst = edge_index[1].reshape(NW, EP)
    pad0 = jnp.zeros((NW, EPP - EP), jnp.int32)
    padN = jnp.full((NW, EPP - EP), N, jnp.int32)
    sg = jnp.concatenate([src, pad0], axis=1).reshape(NW, NCH, CH)
    sd = jnp.concatenate([src, padN], axis=1).reshape(NW, NCH, CH)
    dd = jnp.concatenate([dst, padN], axis=1).reshape(NW, NCH, CH)
    ones8 = jnp.ones((CH, DEGW), jnp.float32)
    z8 = jnp.zeros((NPAD, DEGW), jnp.float32)
    z64 = jnp.zeros((NPAD, H), jnp.float32)

    degp = _deg_call(sd, dd, ones8, z8)
    hn1, ns, nd = _tc_norm_mm1(degp, x, W1)
    agg1 = _agg_call(hn1, sg, dd, z64)
    hn2 = _tc_layer(agg1, nd, ns, b1, W2)
    agg2 = _agg_call(hn2, sg, dd, z64)
    hn3 = _tc_layer(agg2, nd, ns, b2, W3)
    agg3 = _agg_call(hn3, sg, dd, z64)
    return _tc_head(agg3, nd, b3, M1, c1, M2, c2, M3, c3)


# speedup vs baseline: 1.3438x; 1.3438x over previous
import jax
import jax.numpy as jnp
from jax import lax
from jax.experimental import pallas as pl
from jax.experimental.pallas import tpu as pltpu
from jax.experimental.pallas import tpu_sc as plsc

N = 10000
D = 128
H = 64
E = 320000
MLP = 128

NC = 2
NS = 16
NW = NC * NS
EP = E // NW
CH = 128
NBUF = 4
NCH = 79
EPP = NCH * CH
NPAD = 10112
ZR = NPAD // NS
DEGW = 8

_mesh = plsc.VectorSubcoreMesh(core_axis_name="c", subcore_axis_name="s")
_sc_params = pltpu.CompilerParams(use_tc_tiling_on_sc=False)


def _dot3(a, b):
    ah = a.astype(jnp.bfloat16)
    al = (a - ah.astype(jnp.float32)).astype(jnp.bfloat16)
    bh = b.astype(jnp.bfloat16)
    bl = (b - bh.astype(jnp.float32)).astype(jnp.bfloat16)

    def d(u, v):
        return jnp.dot(u, v, preferred_element_type=jnp.float32)

    return d(ah, bh) + d(ah, bl) + d(al, bh)


def _dotd(a, b):
    return jnp.dot(a, b, preferred_element_type=jnp.float32)


def _sc_deg_body(sd_hbm, dd_hbm, ones_hbm, z8_hbm, out_hbm,
                 idx_s, idx_d, ones_v, acc_out, acc_in):
    c = lax.axis_index("c")
    s = lax.axis_index("s")
    w = c * NS + s
    pltpu.sync_copy(sd_hbm.at[w], idx_s)
    pltpu.sync_copy(dd_hbm.at[w], idx_d)
    pltpu.sync_copy(ones_hbm, ones_v)
    pltpu.sync_copy(z8_hbm.at[pl.ds(s * ZR, ZR)], acc_out.at[pl.ds(s * ZR, ZR)])
    pltpu.sync_copy(z8_hbm.at[pl.ds(s * ZR, ZR)], acc_in.at[pl.ds(s * ZR, ZR)])
    plsc.subcore_barrier()

    def body(j, carry):
        pltpu.sync_copy(ones_v, acc_out.at[idx_s.at[j]], add=True)
        pltpu.sync_copy(ones_v, acc_in.at[idx_d.at[j]], add=True)
        return carry

    lax.fori_loop(0, NCH, body, 0)
    plsc.subcore_barrier()
    pltpu.sync_copy(acc_out.at[pl.ds(s * ZR, ZR)],
                    out_hbm.at[c, 0, pl.ds(s * ZR, ZR)])
    pltpu.sync_copy(acc_in.at[pl.ds(s * ZR, ZR)],
                    out_hbm.at[c, 1, pl.ds(s * ZR, ZR)])


_deg_call = pl.kernel(
    _sc_deg_body,
    out_type=jax.ShapeDtypeStruct((NC, 2, NPAD, DEGW), jnp.float32),
    mesh=_mesh,
    scratch_types=[
        pltpu.VMEM((NCH, CH), jnp.int32),
        pltpu.VMEM((NCH, CH), jnp.int32),
        pltpu.VMEM((CH, DEGW), jnp.float32),
        pltpu.VMEM_SHARED((NPAD, DEGW), jnp.float32),
        pltpu.VMEM_SHARED((NPAD, DEGW), jnp.float32),
    ],
    compiler_params=_sc_params,
)


def _sc_agg_body(h_hbm, sg_hbm, dd_hbm, z64_hbm, out_hbm,
                 idx_s, idx_d, gbuf, acc, *sems):
    c = lax.axis_index("c")
    s = lax.axis_index("s")
    w = c * NS + s
    pltpu.sync_copy(sg_hbm.at[w], idx_s)
    pltpu.sync_copy(dd_hbm.at[w], idx_d)
    pltpu.sync_copy(z64_hbm.at[pl.ds(s * ZR, ZR)], acc.at[pl.ds(s * ZR, ZR)])
    plsc.subcore_barrier()

    for b in range(NBUF):
        pltpu.async_copy(h_hbm.at[idx_s.at[b]], gbuf.at[b], sems[b])

    @pl.loop(0, NCH - 3, step=NBUF)
    def _steady(j0):
        for b in range(NBUF):
            j = j0 + b
            pltpu.make_async_copy(h_hbm.at[idx_s.at[j]], gbuf.at[b],
                                  sems[b]).wait()
            pltpu.sync_copy(gbuf.at[b], acc.at[idx_d.at[j]], add=True)

            @pl.when(j + NBUF < NCH)
            def _refill():
                pltpu.async_copy(h_hbm.at[idx_s.at[j + NBUF]], gbuf.at[b],
                                 sems[b])

    for b in range(NCH % NBUF):
        j = NCH - NCH % NBUF + b
        pltpu.make_async_copy(h_hbm.at[idx_s.at[j]], gbuf.at[b],
                              sems[b]).wait()
        pltpu.sync_copy(gbuf.at[b], acc.at[idx_d.at[j]], add=True)

    plsc.subcore_barrier()
    pltpu.sync_copy(acc.at[pl.ds(s * ZR, ZR)], out_hbm.at[c, pl.ds(s * ZR, ZR)])


_agg_call = pl.kernel(
    _sc_agg_body,
    out_type=jax.ShapeDtypeStruct((NC, NPAD, H), jnp.float32),
    mesh=_mesh,
    scratch_types=[
        pltpu.VMEM((NCH, CH), jnp.int32),
        pltpu.VMEM((NCH, CH), jnp.int32),
        pltpu.VMEM((NBUF, CH, H), jnp.float32),
        pltpu.VMEM_SHARED((NPAD, H), jnp.float32),
    ] + [pltpu.SemaphoreType.DMA] * NBUF,
    compiler_params=_sc_params,
)

R = 1000
G = N // R


def _tc_norm_mm1_body(degp_ref, x_ref, w1_ref, hn_ref, ns_ref, nd_ref):
    dp = degp_ref[...]
    deg_out = dp[0, 0, :, :1] + dp[1, 0, :, :1]
    deg_in = dp[0, 1, :, :1] + dp[1, 1, :, :1]
    ns = lax.rsqrt(jnp.maximum(deg_out, 1.0))
    nd = lax.rsqrt(jnp.maximum(deg_in, 1.0))
    h = _dot3(x_ref[...], w1_ref[...])
    hn_ref[...] = h * ns
    ns_ref[...] = ns
    nd_ref[...] = nd


def _tc_norm_mm1(degp, x, W1):
    return pl.pallas_call(
        _tc_norm_mm1_body,
        grid=(G,),
        in_specs=[
            pl.BlockSpec((NC, 2, R, DEGW), lambda i: (0, 0, i, 0)),
            pl.BlockSpec((R, D), lambda i: (i, 0)),
            pl.BlockSpec((D, H), lambda i: (0, 0)),
        ],
        out_specs=[
            pl.BlockSpec((R, H), lambda i: (i, 0)),
            pl.BlockSpec((R, 1), lambda i: (i, 0)),
            pl.BlockSpec((R, 1), lambda i: (i, 0)),
        ],
        out_shape=[
            jax.ShapeDtypeStruct((N, H), jnp.float32),
            jax.ShapeDtypeStruct((N, 1), jnp.float32),
            jax.ShapeDtypeStruct((N, 1), jnp.float32),
        ],
    )(degp, x, W1)


def _tc_layer_body(agg_ref, nd_ref, ns_ref, b_ref, w_ref, out_ref):
    a = agg_ref[0] + agg_ref[1]
    h = jnp.maximum(a * nd_ref[...] + b_ref[...][None, :], 0.0)
    out_ref[...] = _dot3(h, w_ref[...]) * ns_ref[...]


def _tc_layer(agg, nd, ns, b, W):
    return pl.pallas_call(
        _tc_layer_body,
        grid=(G,),
        in_specs=[
            pl.BlockSpec((NC, R, H), lambda i: (0, i, 0)),
            pl.BlockSpec((R, 1), lambda i: (i, 0)),
            pl.BlockSpec((R, 1), lambda i: (i, 0)),
            pl.BlockSpec((H,), lambda i: (0,)),
            pl.BlockSpec((H, H), lambda i: (0, 0)),
        ],
        out_specs=pl.BlockSpec((R, H), lambda i: (i, 0)),
        out_shape=jax.ShapeDtypeStruct((N, H), jnp.float32),
    )(agg, nd, ns, b, W)


def _tc_head_body(agg_ref, nd_ref, b3_ref, m1_ref, c1_ref, m2_ref, c2_ref,
                  m3_ref, c3_ref, out_ref):
    a = agg_ref[0, :N] + agg_ref[1, :N]
    h = jnp.maximum(a * nd_ref[...] + b3_ref[...][None, :], 0.0)
    emb = jnp.sum(h, axis=0, keepdims=True) * (1.0 / N)
    z = jnp.maximum(
        _dot3(emb, m1_ref[...])
        + c1_ref[...][None, :], 0.0)
    z = jnp.maximum(
        _dot3(z, m2_ref[...])
        + c2_ref[...][None, :], 0.0)
    out_ref[...] = (_dot3(z, m3_ref[...])
                    + c3_ref[...][None, :])


def _tc_head(agg, nd, b3, M1, c1, M2, c2, M3, c3):
    return pl.pallas_call(
        _tc_head_body,
        out_shape=jax.ShapeDtypeStruct((1, 1), jnp.float32),
    )(agg, nd, b3, M1, c1, M2, c2, M3, c3)


def kernel(x, W1, b1, W2, b2, W3, b3, M1, c1, M2, c2, M3, c3, edge_index):
    src = edge_index[0].reshape(NW, EP)
    dst = edge_index[1].reshape(NW, EP)
    pad0 = jnp.zeros((NW, EPP - EP), jnp.int32)
    padN = jnp.full((NW, EPP - EP), N, jnp.int32)
    sg = jnp.concatenate([src, pad0], axis=1).reshape(NW, NCH, CH)
    sd = jnp.concatenate([src, padN], axis=1).reshape(NW, NCH, CH)
    dd = jnp.concatenate([dst, padN], axis=1).reshape(NW, NCH, CH)
    ones8 = jnp.ones((CH, DEGW), jnp.float32)
    z8 = jnp.zeros((NPAD, DEGW), jnp.float32)
    z64 = jnp.zeros((NPAD, H), jnp.float32)

    degp = _deg_call(sd, dd, ones8, z8)
    hn1, ns, nd = _tc_norm_mm1(degp, x, W1)
    agg1 = _agg_call(hn1, sg, dd, z64)
    hn2 = _tc_layer(agg1, nd, ns, b1, W2)
    agg2 = _agg_call(hn2, sg, dd, z64)
    hn3 = _tc_layer(agg2, nd, ns, b2, W3)
    agg3 = _agg_call(hn3, sg, dd, z64)
    return _tc_head(agg3, nd, b3, M1, c1, M2, c2, M3, c3)


# speedup vs baseline: 1.3535x; 1.0072x over previous
import jax
import jax.numpy as jnp
from jax import lax
from jax.experimental import pallas as pl
from jax.experimental.pallas import tpu as pltpu
from jax.experimental.pallas import tpu_sc as plsc

N = 10000
D = 128
H = 64
E = 320000
MLP = 128

NC = 2
NS = 16
NW = NC * NS
EP = E // NW
CH = 128
NBUF = 8
NCH = 79
EPP = NCH * CH
NPAD = 10112
ZR = NPAD // NS
DEGW = 8

_mesh = plsc.VectorSubcoreMesh(core_axis_name="c", subcore_axis_name="s")
_sc_params = pltpu.CompilerParams(use_tc_tiling_on_sc=False)


def _dot3(a, b):
    ah = a.astype(jnp.bfloat16)
    al = (a - ah.astype(jnp.float32)).astype(jnp.bfloat16)
    bh = b.astype(jnp.bfloat16)
    bl = (b - bh.astype(jnp.float32)).astype(jnp.bfloat16)

    def d(u, v):
        return jnp.dot(u, v, preferred_element_type=jnp.float32)

    return d(ah, bh) + d(ah, bl) + d(al, bh)


def _dotd(a, b):
    return jnp.dot(a, b, preferred_element_type=jnp.float32)


def _sc_deg_body(sd_hbm, dd_hbm, ones_hbm, z8_hbm, out_hbm,
                 idx_s, idx_d, ones_v, acc_out, acc_in):
    c = lax.axis_index("c")
    s = lax.axis_index("s")
    w = c * NS + s
    pltpu.sync_copy(sd_hbm.at[w], idx_s)
    pltpu.sync_copy(dd_hbm.at[w], idx_d)
    pltpu.sync_copy(ones_hbm, ones_v)
    pltpu.sync_copy(z8_hbm.at[pl.ds(s * ZR, ZR)], acc_out.at[pl.ds(s * ZR, ZR)])
    pltpu.sync_copy(z8_hbm.at[pl.ds(s * ZR, ZR)], acc_in.at[pl.ds(s * ZR, ZR)])
    plsc.subcore_barrier()

    def body(j, carry):
        pltpu.sync_copy(ones_v, acc_out.at[idx_s.at[j]], add=True)
        pltpu.sync_copy(ones_v, acc_in.at[idx_d.at[j]], add=True)
        return carry

    lax.fori_loop(0, NCH, body, 0)
    plsc.subcore_barrier()
    pltpu.sync_copy(acc_out.at[pl.ds(s * ZR, ZR)],
                    out_hbm.at[c, 0, pl.ds(s * ZR, ZR)])
    pltpu.sync_copy(acc_in.at[pl.ds(s * ZR, ZR)],
                    out_hbm.at[c, 1, pl.ds(s * ZR, ZR)])


_deg_call = pl.kernel(
    _sc_deg_body,
    out_type=jax.ShapeDtypeStruct((NC, 2, NPAD, DEGW), jnp.float32),
    mesh=_mesh,
    scratch_types=[
        pltpu.VMEM((NCH, CH), jnp.int32),
        pltpu.VMEM((NCH, CH), jnp.int32),
        pltpu.VMEM((CH, DEGW), jnp.float32),
        pltpu.VMEM_SHARED((NPAD, DEGW), jnp.float32),
        pltpu.VMEM_SHARED((NPAD, DEGW), jnp.float32),
    ],
    compiler_params=_sc_params,
)


def _sc_agg_body(h_hbm, sg_hbm, dd_hbm, z64_hbm, out_hbm,
                 idx_s, idx_d, gbuf, acc, *sems):
    c = lax.axis_index("c")
    s = lax.axis_index("s")
    w = c * NS + s
    pltpu.sync_copy(sg_hbm.at[w], idx_s)
    pltpu.sync_copy(dd_hbm.at[w], idx_d)
    pltpu.sync_copy(z64_hbm.at[pl.ds(s * ZR, ZR)], acc.at[pl.ds(s * ZR, ZR)])
    plsc.subcore_barrier()

    for b in range(NBUF):
        pltpu.async_copy(h_hbm.at[idx_s.at[b]], gbuf.at[b], sems[b])

    @pl.loop(0, NCH - NCH % NBUF, step=NBUF)
    def _steady(j0):
        for b in range(NBUF):
            j = j0 + b
            pltpu.make_async_copy(h_hbm.at[idx_s.at[j]], gbuf.at[b],
                                  sems[b]).wait()
            pltpu.sync_copy(gbuf.at[b], acc.at[idx_d.at[j]], add=True)

            @pl.when(j + NBUF < NCH)
            def _refill():
                pltpu.async_copy(h_hbm.at[idx_s.at[j + NBUF]], gbuf.at[b],
                                 sems[b])

    for b in range(NCH % NBUF):
        j = NCH - NCH % NBUF + b
        pltpu.make_async_copy(h_hbm.at[idx_s.at[j]], gbuf.at[b],
                              sems[b]).wait()
        pltpu.sync_copy(gbuf.at[b], acc.at[idx_d.at[j]], add=True)

    plsc.subcore_barrier()
    pltpu.sync_copy(acc.at[pl.ds(s * ZR, ZR)], out_hbm.at[c, pl.ds(s * ZR, ZR)])


_agg_call = pl.kernel(
    _sc_agg_body,
    out_type=jax.ShapeDtypeStruct((NC, NPAD, H), jnp.float32),
    mesh=_mesh,
    scratch_types=[
        pltpu.VMEM((NCH, CH), jnp.int32),
        pltpu.VMEM((NCH, CH), jnp.int32),
        pltpu.VMEM((NBUF, CH, H), jnp.float32),
        pltpu.VMEM_SHARED((NPAD, H), jnp.float32),
    ] + [pltpu.SemaphoreType.DMA] * NBUF,
    compiler_params=_sc_params,
)

R = 1000
G = N // R


def _tc_norm_mm1_body(degp_ref, x_ref, w1_ref, hn_ref, ns_ref, nd_ref):
    dp = degp_ref[...]
    deg_out = dp[0, 0, :, :1] + dp[1, 0, :, :1]
    deg_in = dp[0, 1, :, :1] + dp[1, 1, :, :1]
    ns = lax.rsqrt(jnp.maximum(deg_out, 1.0))
    nd = lax.rsqrt(jnp.maximum(deg_in, 1.0))
    h = _dot3(x_ref[...], w1_ref[...])
    hn_ref[...] = h * ns
    ns_ref[...] = ns
    nd_ref[...] = nd


def _tc_norm_mm1(degp, x, W1):
    return pl.pallas_call(
        _tc_norm_mm1_body,
        grid=(G,),
        in_specs=[
            pl.BlockSpec((NC, 2, R, DEGW), lambda i: (0, 0, i, 0)),
            pl.BlockSpec((R, D), lambda i: (i, 0)),
            pl.BlockSpec((D, H), lambda i: (0, 0)),
        ],
        out_specs=[
            pl.BlockSpec((R, H), lambda i: (i, 0)),
            pl.BlockSpec((R, 1), lambda i: (i, 0)),
            pl.BlockSpec((R, 1), lambda i: (i, 0)),
        ],
        out_shape=[
            jax.ShapeDtypeStruct((N, H), jnp.float32),
            jax.ShapeDtypeStruct((N, 1), jnp.float32),
            jax.ShapeDtypeStruct((N, 1), jnp.float32),
        ],
    )(degp, x, W1)


def _tc_layer_body(agg_ref, nd_ref, ns_ref, b_ref, w_ref, out_ref):
    a = agg_ref[0] + agg_ref[1]
    h = jnp.maximum(a * nd_ref[...] + b_ref[...][None, :], 0.0)
    out_ref[...] = _dot3(h, w_ref[...]) * ns_ref[...]


def _tc_layer(agg, nd, ns, b, W):
    return pl.pallas_call(
        _tc_layer_body,
        grid=(G,),
        in_specs=[
            pl.BlockSpec((NC, R, H), lambda i: (0, i, 0)),
            pl.BlockSpec((R, 1), lambda i: (i, 0)),
            pl.BlockSpec((R, 1), lambda i: (i, 0)),
            pl.BlockSpec((H,), lambda i: (0,)),
            pl.BlockSpec((H, H), lambda i: (0, 0)),
        ],
        out_specs=pl.BlockSpec((R, H), lambda i: (i, 0)),
        out_shape=jax.ShapeDtypeStruct((N, H), jnp.float32),
    )(agg, nd, ns, b, W)


def _tc_head_body(agg_ref, nd_ref, b3_ref, m1_ref, c1_ref, m2_ref, c2_ref,
                  m3_ref, c3_ref, out_ref):
    a = agg_ref[0, :N] + agg_ref[1, :N]
    h = jnp.maximum(a * nd_ref[...] + b3_ref[...][None, :], 0.0)
    emb = jnp.sum(h, axis=0, keepdims=True) * (1.0 / N)
    z = jnp.maximum(
        _dot3(emb, m1_ref[...])
        + c1_ref[...][None, :], 0.0)
    z = jnp.maximum(
        _dot3(z, m2_ref[...])
        + c2_ref[...][None, :], 0.0)
    out_ref[...] = (_dot3(z, m3_ref[...])
                    + c3_ref[...][None, :])


def _tc_head(agg, nd, b3, M1, c1, M2, c2, M3, c3):
    return pl.pallas_call(
        _tc_head_body,
        out_shape=jax.ShapeDtypeStruct((1, 1), jnp.float32),
    )(agg, nd, b3, M1, c1, M2, c2, M3, c3)


def kernel(x, W1, b1, W2, b2, W3, b3, M1, c1, M2, c2, M3, c3, edge_index):
    src = edge_index[0].reshape(NW, EP)
    dst = edge_index[1].reshape(NW, EP)
    pad0 = jnp.zeros((NW, EPP - EP), jnp.int32)
    padN = jnp.full((NW, EPP - EP), N, jnp.int32)
    sg = jnp.concatenate([src, pad0], axis=1).reshape(NW, NCH, CH)
    sd = jnp.concatenate([src, padN], axis=1).reshape(NW, NCH, CH)
    dd = jnp.concatenate([dst, padN], axis=1).reshape(NW, NCH, CH)
    ones8 = jnp.ones((CH, DEGW), jnp.float32)
    z8 = jnp.zeros((NPAD, DEGW), jnp.float32)
    z64 = jnp.zeros((NPAD, H), jnp.float32)

    degp = _deg_call(sd, dd, ones8, z8)
    hn1, ns, nd = _tc_norm_mm1(degp, x, W1)
    agg1 = _agg_call(hn1, sg, dd, z64)
    hn2 = _tc_layer(agg1, nd, ns, b1, W2)
    agg2 = _agg_call(hn2, sg, dd, z64)
    hn3 = _tc_layer(agg2, nd, ns, b2, W3)
    agg3 = _agg_call(hn3, sg, dd, z64)
    return _tc_head(agg3, nd, b3, M1, c1, M2, c2, M3, c3)
